# baseline (speedup 1.0000x reference)
import functools

import jax
import jax.numpy as jnp
from jax import lax
from jax.experimental import pallas as pl
from jax.experimental.pallas import tpu as pltpu
from jax.experimental.pallas import tpu_sc as plsc

_EMB = 256
_HID = 256
_L = 512
_B = 4
_T = 3600
_TP = 3648
_PW = 456
_NW = 32
_ZERO_ROW = _B * _L
_LAST_SKIP = _TP - _T


def _layer_norm(h, g, b):
    mu = jnp.mean(h, axis=1, keepdims=True)
    d = h - mu
    var = jnp.mean(d * d, axis=1, keepdims=True)
    return d * lax.rsqrt(var + 1e-5) * g + b


def _shift(h, z):
    prev = jnp.concatenate([z, h[:-1, :]], axis=0)
    nxt = jnp.concatenate([h[1:, :], z], axis=0)
    return prev, nxt


def _tc_body(mel_ref, x_ref, tgt_ref, wk1_ref, b1_ref, g1_ref, be1_ref,
             wk2_ref, b2_ref, g2_ref, be2_ref, lw_ref, lb_ref,
             dur_ref, idx_ref):
    b = pl.program_id(0)
    xb = x_ref[0]

    z = jnp.zeros((1, _EMB), jnp.float32)
    xp, xn = _shift(xb, z)
    h = (jnp.dot(xp, wk1_ref[0], preferred_element_type=jnp.float32)
         + jnp.dot(xb, wk1_ref[1], preferred_element_type=jnp.float32)
         + jnp.dot(xn, wk1_ref[2], preferred_element_type=jnp.float32)
         + b1_ref[...])
    h = jax.nn.relu(_layer_norm(h, g1_ref[...], be1_ref[...]))
    hp, hn = _shift(h, z)
    h = (jnp.dot(hp, wk2_ref[0], preferred_element_type=jnp.float32)
         + jnp.dot(h, wk2_ref[1], preferred_element_type=jnp.float32)
         + jnp.dot(hn, wk2_ref[2], preferred_element_type=jnp.float32)
         + b2_ref[...])
    h = jax.nn.relu(_layer_norm(h, g2_ref[...], be2_ref[...]))
    dur = jax.nn.relu(jnp.dot(h, lw_ref[...],
                              preferred_element_type=jnp.float32)
                      + lb_ref[...])
    dur_ref[0] = dur

    rr = lax.broadcasted_iota(jnp.int32, (_L, _L), 0)
    cc = lax.broadcasted_iota(jnp.int32, (_L, _L), 1)
    ut = (rr <= cc).astype(jnp.float32)
    tgt_row = tgt_ref[0].astype(jnp.float32)
    ends_row = jnp.dot(tgt_row, ut,
                       preferred_element_type=jnp.float32)
    ends_i = ends_row.astype(jnp.int32)
    total = ends_i[0, _L - 1]
    limit = jnp.minimum(total, mel_ref[0, 0])
    base = b * _L
    nblk = 4
    blk = _TP // nblk
    for k in range(nblk):
        t_col = lax.broadcasted_iota(jnp.int32, (blk, 1), 0) + k * blk
        cnt = jnp.sum((ends_i <= t_col).astype(jnp.int32), axis=1,
                      keepdims=True)
        absidx = jnp.where(t_col < limit, cnt + base, _ZERO_ROW)
        idx_ref[0, pl.ds(k * blk, blk)] = absidx


def _run_tc(x, tgt, mel, wk1, b1r, g1r, be1r, wk2, b2r, g2r, be2r, lw, lbr):
    full3 = lambda s: pl.BlockSpec(s, lambda b: (0, 0, 0))
    full2 = lambda s: pl.BlockSpec(s, lambda b: (0, 0))
    return pl.pallas_call(
        _tc_body,
        grid=(_B,),
        in_specs=[
            pl.BlockSpec(memory_space=pltpu.SMEM),
            pl.BlockSpec((1, _L, _EMB), lambda b: (b, 0, 0)),
            pl.BlockSpec((1, 1, _L), lambda b: (b, 0, 0)),
            full3((3, _EMB, _HID)),
            full2((1, _HID)), full2((1, _HID)), full2((1, _HID)),
            full3((3, _HID, _HID)),
            full2((1, _HID)), full2((1, _HID)), full2((1, _HID)),
            full2((_HID, 1)),
            full2((1, 1)),
        ],
        out_specs=[
            pl.BlockSpec((1, _L, 1), lambda b: (b, 0, 0)),
            pl.BlockSpec((1, _TP, 1), lambda b: (b, 0, 0)),
        ],
        out_shape=[
            jax.ShapeDtypeStruct((_B, _L, 1), jnp.float32),
            jax.ShapeDtypeStruct((_B, _TP, 1), jnp.int32),
        ],
    )(mel, x, tgt, wk1, b1r, g1r, be1r, wk2, b2r, g2r, be2r, lw, lbr)


def _sc_body(table_hbm, idx_hbm, out_hbm, idxv, sem):
    cid = lax.axis_index("c")
    sid = lax.axis_index("s")
    wid = cid * 16 + sid
    b = wid // 8
    slot = wid % 8
    pad_base = wid * _PW
    out_base = b * _T + slot * _PW
    pltpu.sync_copy(idx_hbm.at[pl.ds(pad_base, _PW)], idxv.at[pl.ds(0, _PW)])
    nvalid = jnp.where(slot == 7, _PW - _LAST_SKIP, _PW)

    def body(g, carry):
        v = idxv[pl.ds(g * 16, 16)]
        for j in range(16):
            r = g * 16 + j

            @pl.when(r < nvalid)
            def _():
                pltpu.async_copy(table_hbm.at[pl.ds(v[j], 1)],
                                 out_hbm.at[pl.ds(out_base + r, 1)], sem)
        return carry

    lax.fori_loop(0, (nvalid + 15) // 16, body, 0)

    @pl.when(slot == 7)
    def _():
        pltpu.make_async_copy(
            out_hbm.at[pl.ds(out_base, _PW - _LAST_SKIP)],
            out_hbm.at[pl.ds(out_base, _PW - _LAST_SKIP)], sem).wait()

    @pl.when(slot != 7)
    def _():
        pltpu.make_async_copy(
            out_hbm.at[pl.ds(out_base, _PW)],
            out_hbm.at[pl.ds(out_base, _PW)], sem).wait()


def _run_sc(table, flat_idx):
    mesh = plsc.VectorSubcoreMesh(core_axis_name="c", subcore_axis_name="s")
    f = functools.partial(
        pl.kernel,
        out_type=jax.ShapeDtypeStruct((_B * _T, _EMB), jnp.float32),
        mesh=mesh,
        scratch_types=[
            pltpu.VMEM((464,), jnp.int32),
            pltpu.SemaphoreType.DMA,
        ],
    )(_sc_body)
    return f(table, flat_idx)


def kernel(x, target, mel_max_len, conv1_w, conv1_b, ln1_g, ln1_b,
           conv2_w, conv2_b, ln2_g, ln2_b, lin_w, lin_b):
    mel = jnp.asarray(mel_max_len, jnp.int32).reshape(1, 1)
    tgt = target.astype(jnp.int32).reshape(_B, 1, _L)
    wk1 = jnp.transpose(conv1_w, (2, 1, 0))
    wk2 = jnp.transpose(conv2_w, (2, 1, 0))
    dur3, idx3 = _run_tc(
        x, tgt, mel, wk1,
        conv1_b.reshape(1, _HID), ln1_g.reshape(1, _HID), ln1_b.reshape(1, _HID),
        wk2,
        conv2_b.reshape(1, _HID), ln2_g.reshape(1, _HID), ln2_b.reshape(1, _HID),
        lin_w, lin_b.reshape(1, 1))
    duration = dur3[:, :, 0]
    table = jnp.concatenate(
        [x.reshape(_B * _L, _EMB), jnp.zeros((8, _EMB), x.dtype)], axis=0)
    flat_idx = idx3.reshape(_B * _TP)
    out_flat = _run_sc(table, flat_idx)
    output = out_flat.reshape(_B, _T, _EMB)
    return (output, duration)


# speedup vs baseline: 12.1942x; 12.1942x over previous
import jax
import jax.numpy as jnp
from jax import lax
from jax.experimental import pallas as pl
from jax.experimental.pallas import tpu as pltpu

_EMB = 256
_HID = 256
_L = 512
_B = 4
_T = 3600
_TB = 720
_NT = _T // _TB


def _layer_norm(h, g, b):
    mu = jnp.mean(h, axis=1, keepdims=True)
    d = h - mu
    var = jnp.mean(d * d, axis=1, keepdims=True)
    return d * lax.rsqrt(var + 1e-5) * g + b


def _shift(h, z):
    prev = jnp.concatenate([z, h[:-1, :]], axis=0)
    nxt = jnp.concatenate([h[1:, :], z], axis=0)
    return prev, nxt


def _dur_body(x_ref, wk1_ref, b1_ref, g1_ref, be1_ref,
              wk2_ref, b2_ref, g2_ref, be2_ref, lw_ref, lb_ref, dur_ref):
    xb = x_ref[0]
    z = jnp.zeros((1, _EMB), jnp.float32)
    xp, xn = _shift(xb, z)
    h = (jnp.dot(xp, wk1_ref[0], preferred_element_type=jnp.float32)
         + jnp.dot(xb, wk1_ref[1], preferred_element_type=jnp.float32)
         + jnp.dot(xn, wk1_ref[2], preferred_element_type=jnp.float32)
         + b1_ref[...])
    h = jax.nn.relu(_layer_norm(h, g1_ref[...], be1_ref[...]))
    hp, hn = _shift(h, z)
    h = (jnp.dot(hp, wk2_ref[0], preferred_element_type=jnp.float32)
         + jnp.dot(h, wk2_ref[1], preferred_element_type=jnp.float32)
         + jnp.dot(hn, wk2_ref[2], preferred_element_type=jnp.float32)
         + b2_ref[...])
    h = jax.nn.relu(_layer_norm(h, g2_ref[...], be2_ref[...]))
    dur = jax.nn.relu(jnp.dot(h, lw_ref[...],
                              preferred_element_type=jnp.float32)
                      + lb_ref[...])
    dur_ref[0] = dur


def _run_dur(x, wk1, b1r, g1r, be1r, wk2, b2r, g2r, be2r, lw, lbr):
    full3 = lambda s: pl.BlockSpec(s, lambda b: (0, 0, 0))
    full2 = lambda s: pl.BlockSpec(s, lambda b: (0, 0))
    return pl.pallas_call(
        _dur_body,
        grid=(_B,),
        in_specs=[
            pl.BlockSpec((1, _L, _EMB), lambda b: (b, 0, 0)),
            full3((3, _EMB, _HID)),
            full2((1, _HID)), full2((1, _HID)), full2((1, _HID)),
            full3((3, _HID, _HID)),
            full2((1, _HID)), full2((1, _HID)), full2((1, _HID)),
            full2((_HID, 1)),
            full2((1, 1)),
        ],
        out_specs=pl.BlockSpec((1, _L, 1), lambda b: (b, 0, 0)),
        out_shape=jax.ShapeDtypeStruct((_B, _L, 1), jnp.float32),
    )(x, wk1, b1r, g1r, be1r, wk2, b2r, g2r, be2r, lw, lbr)


def _lr_body(mel_ref, tgt_ref, xbf_ref, out_ref):
    jt = pl.program_id(1)
    rr = lax.broadcasted_iota(jnp.int32, (_L, _L), 0)
    cc = lax.broadcasted_iota(jnp.int32, (_L, _L), 1)
    ut = (rr <= cc).astype(jnp.float32)
    tgt_row = tgt_ref[0].astype(jnp.float32)
    ends_row = jnp.dot(tgt_row, ut,
                       preferred_element_type=jnp.float32)
    ends_i = ends_row.astype(jnp.int32)
    starts_i = ends_i - tgt_ref[0]
    ends_eff = jnp.minimum(ends_i, mel_ref[0, 0])
    t_col = lax.broadcasted_iota(jnp.int32, (_TB, 1), 0) + jt * _TB
    a_blk = ((starts_i <= t_col) & (t_col < ends_eff)).astype(jnp.bfloat16)
    out_ref[0] = jnp.dot(a_blk, xbf_ref[0],
                         preferred_element_type=jnp.float32)


def _run_lr(mel, tgt3, xbf):
    return pl.pallas_call(
        _lr_body,
        grid=(_B, _NT),
        in_specs=[
            pl.BlockSpec(memory_space=pltpu.SMEM),
            pl.BlockSpec((1, 1, _L), lambda b, jt: (b, 0, 0)),
            pl.BlockSpec((1, _L, _EMB), lambda b, jt: (b, 0, 0)),
        ],
        out_specs=pl.BlockSpec((1, _TB, _EMB), lambda b, jt: (b, jt, 0)),
        out_shape=jax.ShapeDtypeStruct((_B, _T, _EMB), jnp.float32),
    )(mel, tgt3, xbf)


def kernel(x, target, mel_max_len, conv1_w, conv1_b, ln1_g, ln1_b,
           conv2_w, conv2_b, ln2_g, ln2_b, lin_w, lin_b):
    mel = jnp.asarray(mel_max_len, jnp.int32).reshape(1, 1)
    tgt3 = target.astype(jnp.int32).reshape(_B, 1, _L)
    wk1 = jnp.transpose(conv1_w, (2, 1, 0))
    wk2 = jnp.transpose(conv2_w, (2, 1, 0))
    dur3 = _run_dur(
        x, wk1,
        conv1_b.reshape(1, _HID), ln1_g.reshape(1, _HID), ln1_b.reshape(1, _HID),
        wk2,
        conv2_b.reshape(1, _HID), ln2_g.reshape(1, _HID), ln2_b.reshape(1, _HID),
        lin_w, lin_b.reshape(1, 1))
    output = _run_lr(mel, tgt3, x.astype(jnp.bfloat16))
    return (output, dur3[:, :, 0])


# speedup vs baseline: 13.0344x; 1.0689x over previous
import jax
import jax.numpy as jnp
from jax import lax
from jax.experimental import pallas as pl
from jax.experimental.pallas import tpu as pltpu

_EMB = 256
_HID = 256
_L = 512
_B = 4
_T = 3600
_TB = 720
_NT = _T // _TB
_W = 256


def _layer_norm(h, g, b):
    mu = jnp.mean(h, axis=1, keepdims=True)
    d = h - mu
    var = jnp.mean(d * d, axis=1, keepdims=True)
    return d * lax.rsqrt(var + 1e-5) * g + b


def _shift(h, z):
    prev = jnp.concatenate([z, h[:-1, :]], axis=0)
    nxt = jnp.concatenate([h[1:, :], z], axis=0)
    return prev, nxt


def _dur_body(x_ref, tgt_ref, wk1_ref, b1_ref, g1_ref, be1_ref,
              wk2_ref, b2_ref, g2_ref, be2_ref, lw_ref, lb_ref,
              dur_ref, st_ref, en_ref, xbf_ref):
    xb = x_ref[0]
    xbf_ref[0] = xb.astype(jnp.bfloat16)
    z = jnp.zeros((1, _EMB), jnp.float32)
    xp, xn = _shift(xb, z)
    h = (jnp.dot(xp, wk1_ref[0], preferred_element_type=jnp.float32)
         + jnp.dot(xb, wk1_ref[1], preferred_element_type=jnp.float32)
         + jnp.dot(xn, wk1_ref[2], preferred_element_type=jnp.float32)
         + b1_ref[...])
    h = jax.nn.relu(_layer_norm(h, g1_ref[...], be1_ref[...]))
    hp, hn = _shift(h, z)
    h = (jnp.dot(hp, wk2_ref[0], preferred_element_type=jnp.float32)
         + jnp.dot(h, wk2_ref[1], preferred_element_type=jnp.float32)
         + jnp.dot(hn, wk2_ref[2], preferred_element_type=jnp.float32)
         + b2_ref[...])
    h = jax.nn.relu(_layer_norm(h, g2_ref[...], be2_ref[...]))
    dur = jax.nn.relu(jnp.dot(h, lw_ref[...],
                              preferred_element_type=jnp.float32)
                      + lb_ref[...])
    dur_ref[0] = dur

    rr = lax.broadcasted_iota(jnp.int32, (_L, _L), 0)
    cc = lax.broadcasted_iota(jnp.int32, (_L, _L), 1)
    ut = (rr <= cc).astype(jnp.float32)
    tgt_row = tgt_ref[0].astype(jnp.float32)
    ends_row = jnp.dot(tgt_row, ut,
                       preferred_element_type=jnp.float32)
    ends_i = ends_row.astype(jnp.int32)
    en_ref[0] = ends_i
    st_ref[0] = ends_i - tgt_ref[0]


def _run_dur(x, tgt3, wk1, b1r, g1r, be1r, wk2, b2r, g2r, be2r, lw, lbr):
    full3 = lambda s: pl.BlockSpec(s, lambda b: (0, 0, 0))
    full2 = lambda s: pl.BlockSpec(s, lambda b: (0, 0))
    return pl.pallas_call(
        _dur_body,
        grid=(_B,),
        in_specs=[
            pl.BlockSpec((1, _L, _EMB), lambda b: (b, 0, 0)),
            pl.BlockSpec((1, 1, _L), lambda b: (b, 0, 0)),
            full3((3, _EMB, _HID)),
            full2((1, _HID)), full2((1, _HID)), full2((1, _HID)),
            full3((3, _HID, _HID)),
            full2((1, _HID)), full2((1, _HID)), full2((1, _HID)),
            full2((_HID, 1)),
            full2((1, 1)),
        ],
        out_specs=[
            pl.BlockSpec((1, _L, 1), lambda b: (b, 0, 0)),
            pl.BlockSpec((1, 1, _L), lambda b: (b, 0, 0)),
            pl.BlockSpec((1, 1, _L), lambda b: (b, 0, 0)),
            pl.BlockSpec((1, _L, _EMB), lambda b: (b, 0, 0)),
        ],
        out_shape=[
            jax.ShapeDtypeStruct((_B, _L, 1), jnp.float32),
            jax.ShapeDtypeStruct((_B, 1, _L), jnp.int32),
            jax.ShapeDtypeStruct((_B, 1, _L), jnp.int32),
            jax.ShapeDtypeStruct((_B, _L, _EMB), jnp.bfloat16),
        ],
    )(x, tgt3, wk1, b1r, g1r, be1r, wk2, b2r, g2r, be2r, lw, lbr)


def _lr_body(mel_ref, st_ref, en_ref, xbf_ref, out_ref):
    jt = pl.program_id(1)
    mel = mel_ref[0, 0]
    starts_i = st_ref[0]
    ends_i = en_ref[0]
    t0 = jt * _TB
    t1 = t0 + _TB
    lo = jnp.sum((ends_i <= t0).astype(jnp.int32))
    hi = jnp.sum((starts_i < t1).astype(jnp.int32))
    shift = jnp.remainder(_L - lo, _L)
    t_col = lax.broadcasted_iota(jnp.int32, (_TB, 1), 0) + t0

    @pl.when(hi - lo <= _W)
    def _():
        st_w = pltpu.roll(starts_i, shift, axis=1)[:, :_W]
        en_w = jnp.minimum(pltpu.roll(ends_i, shift, axis=1)[:, :_W], mel)
        a_blk = ((st_w <= t_col) & (t_col < en_w)).astype(jnp.bfloat16)
        x_w = pltpu.roll(xbf_ref[0], shift, axis=0)[:_W, :]
        out_ref[0] = jnp.dot(a_blk, x_w, preferred_element_type=jnp.float32)

    @pl.when(hi - lo > _W)
    def _():
        ends_eff = jnp.minimum(ends_i, mel)
        a_blk = ((starts_i <= t_col) & (t_col < ends_eff)).astype(jnp.bfloat16)
        out_ref[0] = jnp.dot(a_blk, xbf_ref[0],
                             preferred_element_type=jnp.float32)


def _run_lr(mel, st3, en3, xbf):
    return pl.pallas_call(
        _lr_body,
        grid=(_B, _NT),
        in_specs=[
            pl.BlockSpec(memory_space=pltpu.SMEM),
            pl.BlockSpec((1, 1, _L), lambda b, jt: (b, 0, 0)),
            pl.BlockSpec((1, 1, _L), lambda b, jt: (b, 0, 0)),
            pl.BlockSpec((1, _L, _EMB), lambda b, jt: (b, 0, 0)),
        ],
        out_specs=pl.BlockSpec((1, _TB, _EMB), lambda b, jt: (b, jt, 0)),
        out_shape=jax.ShapeDtypeStruct((_B, _T, _EMB), jnp.float32),
    )(mel, st3, en3, xbf)


def kernel(x, target, mel_max_len, conv1_w, conv1_b, ln1_g, ln1_b,
           conv2_w, conv2_b, ln2_g, ln2_b, lin_w, lin_b):
    mel = jnp.asarray(mel_max_len, jnp.int32).reshape(1, 1)
    tgt3 = target.astype(jnp.int32).reshape(_B, 1, _L)
    wk1 = jnp.transpose(conv1_w, (2, 1, 0))
    wk2 = jnp.transpose(conv2_w, (2, 1, 0))
    dur3, st3, en3, xbf = _run_dur(
        x, tgt3, wk1,
        conv1_b.reshape(1, _HID), ln1_g.reshape(1, _HID), ln1_b.reshape(1, _HID),
        wk2,
        conv2_b.reshape(1, _HID), ln2_g.reshape(1, _HID), ln2_b.reshape(1, _HID),
        lin_w, lin_b.reshape(1, 1))
    output = _run_lr(mel, st3, en3, xbf)
    return (output, dur3[:, :, 0])


# speedup vs baseline: 13.5634x; 1.0406x over previous
import jax
import jax.numpy as jnp
from jax import lax
from jax.experimental import pallas as pl
from jax.experimental.pallas import tpu as pltpu

_EMB = 256
_HID = 256
_L = 512
_B = 4
_T = 3600
_TB = 720
_NT = _T // _TB
_W = 256


def _layer_norm(h, g, b):
    mu = jnp.mean(h, axis=1, keepdims=True)
    d = h - mu
    var = jnp.mean(d * d, axis=1, keepdims=True)
    return d * lax.rsqrt(var + 1e-5) * g + b


def _shift(h, z):
    prev = jnp.concatenate([z, h[:-1, :]], axis=0)
    nxt = jnp.concatenate([h[1:, :], z], axis=0)
    return prev, nxt


def _body(mel_ref, x_ref, tgt_ref, wk1_ref, b1_ref, g1_ref, be1_ref,
          wk2_ref, b2_ref, g2_ref, be2_ref, lw_ref, lb_ref,
          out_ref, dur_ref, st_s, en_s, xbf_s):
    jt = pl.program_id(1)

    @pl.when(jt == 0)
    def _():
        xb = x_ref[0]
        xbf_s[...] = xb.astype(jnp.bfloat16)
        z = jnp.zeros((1, _EMB), jnp.float32)
        xp, xn = _shift(xb, z)
        h = (jnp.dot(xp, wk1_ref[0], preferred_element_type=jnp.float32)
             + jnp.dot(xb, wk1_ref[1], preferred_element_type=jnp.float32)
             + jnp.dot(xn, wk1_ref[2], preferred_element_type=jnp.float32)
             + b1_ref[...])
        h = jax.nn.relu(_layer_norm(h, g1_ref[...], be1_ref[...]))
        hp, hn = _shift(h, z)
        h = (jnp.dot(hp, wk2_ref[0], preferred_element_type=jnp.float32)
             + jnp.dot(h, wk2_ref[1], preferred_element_type=jnp.float32)
             + jnp.dot(hn, wk2_ref[2], preferred_element_type=jnp.float32)
             + b2_ref[...])
        h = jax.nn.relu(_layer_norm(h, g2_ref[...], be2_ref[...]))
        dur = jax.nn.relu(jnp.dot(h, lw_ref[...],
                                  preferred_element_type=jnp.float32)
                          + lb_ref[...])
        dur_ref[0] = dur

        rr = lax.broadcasted_iota(jnp.int32, (_L, _L), 0)
        cc = lax.broadcasted_iota(jnp.int32, (_L, _L), 1)
        ut = (rr <= cc).astype(jnp.float32)
        tgt_row = tgt_ref[0].astype(jnp.float32)
        ends_row = jnp.dot(tgt_row, ut,
                           preferred_element_type=jnp.float32)
        ends_i = ends_row.astype(jnp.int32)
        en_s[...] = ends_i
        st_s[...] = ends_i - tgt_ref[0]

    mel = mel_ref[0, 0]
    starts_i = st_s[...]
    ends_i = en_s[...]
    t0 = jt * _TB
    t1 = t0 + _TB
    lo = jnp.sum((ends_i <= t0).astype(jnp.int32))
    hi = jnp.sum((starts_i < t1).astype(jnp.int32))
    shift = jnp.remainder(_L - lo, _L)
    t_col = lax.broadcasted_iota(jnp.int32, (_TB, 1), 0) + t0

    @pl.when(hi - lo <= _W)
    def _():
        st_w = pltpu.roll(starts_i, shift, axis=1)[:, :_W]
        en_w = jnp.minimum(pltpu.roll(ends_i, shift, axis=1)[:, :_W], mel)
        a_blk = ((st_w <= t_col) & (t_col < en_w)).astype(jnp.bfloat16)
        x_w = pltpu.roll(xbf_s[...], shift, axis=0)[:_W, :]
        out_ref[0] = jnp.dot(a_blk, x_w, preferred_element_type=jnp.float32)

    @pl.when(hi - lo > _W)
    def _():
        ends_eff = jnp.minimum(ends_i, mel)
        a_blk = ((starts_i <= t_col) & (t_col < ends_eff)).astype(jnp.bfloat16)
        out_ref[0] = jnp.dot(a_blk, xbf_s[...],
                             preferred_element_type=jnp.float32)


def kernel(x, target, mel_max_len, conv1_w, conv1_b, ln1_g, ln1_b,
           conv2_w, conv2_b, ln2_g, ln2_b, lin_w, lin_b):
    mel = jnp.asarray(mel_max_len, jnp.int32).reshape(1, 1)
    tgt3 = target.astype(jnp.int32).reshape(_B, 1, _L)
    wk1 = jnp.transpose(conv1_w, (2, 1, 0))
    wk2 = jnp.transpose(conv2_w, (2, 1, 0))
    full3 = lambda s: pl.BlockSpec(s, lambda b, jt: (0, 0, 0))
    full2 = lambda s: pl.BlockSpec(s, lambda b, jt: (0, 0))
    output, dur3 = pl.pallas_call(
        _body,
        grid=(_B, _NT),
        in_specs=[
            pl.BlockSpec(memory_space=pltpu.SMEM),
            pl.BlockSpec((1, _L, _EMB), lambda b, jt: (b, 0, 0)),
            pl.BlockSpec((1, 1, _L), lambda b, jt: (b, 0, 0)),
            full3((3, _EMB, _HID)),
            full2((1, _HID)), full2((1, _HID)), full2((1, _HID)),
            full3((3, _HID, _HID)),
            full2((1, _HID)), full2((1, _HID)), full2((1, _HID)),
            full2((_HID, 1)),
            full2((1, 1)),
        ],
        out_specs=[
            pl.BlockSpec((1, _TB, _EMB), lambda b, jt: (b, jt, 0)),
            pl.BlockSpec((1, _L, 1), lambda b, jt: (b, 0, 0)),
        ],
        out_shape=[
            jax.ShapeDtypeStruct((_B, _T, _EMB), jnp.float32),
            jax.ShapeDtypeStruct((_B, _L, 1), jnp.float32),
        ],
        scratch_shapes=[
            pltpu.VMEM((1, _L), jnp.int32),
            pltpu.VMEM((1, _L), jnp.int32),
            pltpu.VMEM((_L, _EMB), jnp.bfloat16),
        ],
    )(mel, x, tgt3, wk1,
      conv1_b.reshape(1, _HID), ln1_g.reshape(1, _HID), ln1_b.reshape(1, _HID),
      wk2,
      conv2_b.reshape(1, _HID), ln2_g.reshape(1, _HID), ln2_b.reshape(1, _HID),
      lin_w, lin_b.reshape(1, 1))
    return (output, dur3[:, :, 0])


# speedup vs baseline: 15.0070x; 1.1064x over previous
import jax
import jax.numpy as jnp
from jax import lax
from jax.experimental import pallas as pl
from jax.experimental.pallas import tpu as pltpu

_EMB = 256
_HID = 256
_L = 512
_B = 4
_T = 3600
_TB = 1200
_NT = _T // _TB
_W = 256


def _layer_norm(h, g, b):
    mu = jnp.mean(h, axis=1, keepdims=True)
    d = h - mu
    var = jnp.mean(d * d, axis=1, keepdims=True)
    return d * lax.rsqrt(var + 1e-5) * g + b


def _shift(h, z):
    prev = jnp.concatenate([z, h[:-1, :]], axis=0)
    nxt = jnp.concatenate([h[1:, :], z], axis=0)
    return prev, nxt


def _body(mel_ref, x_ref, tgt_ref, wk1_ref, b1_ref, g1_ref, be1_ref,
          wk2_ref, b2_ref, g2_ref, be2_ref, lw_ref, lb_ref,
          out_ref, dur_ref, st_s, en_s, xbf_s):
    jt = pl.program_id(1)

    @pl.when(jt == 0)
    def _():
        xb = x_ref[0]
        xbf_s[...] = xb.astype(jnp.bfloat16)
        z = jnp.zeros((1, _EMB), jnp.float32)
        xp, xn = _shift(xb, z)
        h = (jnp.dot(xp, wk1_ref[0], preferred_element_type=jnp.float32)
             + jnp.dot(xb, wk1_ref[1], preferred_element_type=jnp.float32)
             + jnp.dot(xn, wk1_ref[2], preferred_element_type=jnp.float32)
             + b1_ref[...])
        h = jax.nn.relu(_layer_norm(h, g1_ref[...], be1_ref[...]))
        hp, hn = _shift(h, z)
        h = (jnp.dot(hp, wk2_ref[0], preferred_element_type=jnp.float32)
             + jnp.dot(h, wk2_ref[1], preferred_element_type=jnp.float32)
             + jnp.dot(hn, wk2_ref[2], preferred_element_type=jnp.float32)
             + b2_ref[...])
        h = jax.nn.relu(_layer_norm(h, g2_ref[...], be2_ref[...]))
        dur = jax.nn.relu(jnp.dot(h, lw_ref[...],
                                  preferred_element_type=jnp.float32)
                          + lb_ref[...])
        dur_ref[0] = dur

        rr = lax.broadcasted_iota(jnp.int32, (_L, _L), 0)
        cc = lax.broadcasted_iota(jnp.int32, (_L, _L), 1)
        ut = (rr <= cc).astype(jnp.float32)
        tgt_row = tgt_ref[0].astype(jnp.float32)
        ends_row = jnp.dot(tgt_row, ut,
                           preferred_element_type=jnp.float32)
        ends_i = ends_row.astype(jnp.int32)
        en_s[...] = ends_i
        st_s[...] = ends_i - tgt_ref[0]

    mel = mel_ref[0, 0]
    starts_i = st_s[...]
    ends_i = en_s[...]
    t0 = jt * _TB
    t1 = t0 + _TB
    lo = jnp.sum((ends_i <= t0).astype(jnp.int32))
    hi = jnp.sum((starts_i < t1).astype(jnp.int32))
    shift = jnp.remainder(_L - lo, _L)
    t_col = lax.broadcasted_iota(jnp.int32, (_TB, 1), 0) + t0

    @pl.when(hi - lo <= _W)
    def _():
        st_w = pltpu.roll(starts_i, shift, axis=1)[:, :_W]
        en_w = jnp.minimum(pltpu.roll(ends_i, shift, axis=1)[:, :_W], mel)
        a_blk = ((st_w <= t_col) & (t_col < en_w)).astype(jnp.bfloat16)
        x_w = pltpu.roll(xbf_s[...], shift, axis=0)[:_W, :]
        out_ref[0] = jnp.dot(a_blk, x_w, preferred_element_type=jnp.float32)

    @pl.when(hi - lo > _W)
    def _():
        ends_eff = jnp.minimum(ends_i, mel)
        a_blk = ((starts_i <= t_col) & (t_col < ends_eff)).astype(jnp.bfloat16)
        out_ref[0] = jnp.dot(a_blk, xbf_s[...],
                             preferred_element_type=jnp.float32)


def kernel(x, target, mel_max_len, conv1_w, conv1_b, ln1_g, ln1_b,
           conv2_w, conv2_b, ln2_g, ln2_b, lin_w, lin_b):
    mel = jnp.asarray(mel_max_len, jnp.int32).reshape(1, 1)
    tgt3 = target.astype(jnp.int32).reshape(_B, 1, _L)
    wk1 = jnp.transpose(conv1_w, (2, 1, 0))
    wk2 = jnp.transpose(conv2_w, (2, 1, 0))
    full3 = lambda s: pl.BlockSpec(s, lambda b, jt: (0, 0, 0))
    full2 = lambda s: pl.BlockSpec(s, lambda b, jt: (0, 0))
    output, dur3 = pl.pallas_call(
        _body,
        grid=(_B, _NT),
        in_specs=[
            pl.BlockSpec(memory_space=pltpu.SMEM),
            pl.BlockSpec((1, _L, _EMB), lambda b, jt: (b, 0, 0)),
            pl.BlockSpec((1, 1, _L), lambda b, jt: (b, 0, 0)),
            full3((3, _EMB, _HID)),
            full2((1, _HID)), full2((1, _HID)), full2((1, _HID)),
            full3((3, _HID, _HID)),
            full2((1, _HID)), full2((1, _HID)), full2((1, _HID)),
            full2((_HID, 1)),
            full2((1, 1)),
        ],
        out_specs=[
            pl.BlockSpec((1, _TB, _EMB), lambda b, jt: (b, jt, 0)),
            pl.BlockSpec((1, _L, 1), lambda b, jt: (b, 0, 0)),
        ],
        out_shape=[
            jax.ShapeDtypeStruct((_B, _T, _EMB), jnp.float32),
            jax.ShapeDtypeStruct((_B, _L, 1), jnp.float32),
        ],
        scratch_shapes=[
            pltpu.VMEM((1, _L), jnp.int32),
            pltpu.VMEM((1, _L), jnp.int32),
            pltpu.VMEM((_L, _EMB), jnp.bfloat16),
        ],
    )(mel, x, tgt3, wk1,
      conv1_b.reshape(1, _HID), ln1_g.reshape(1, _HID), ln1_b.reshape(1, _HID),
      wk2,
      conv2_b.reshape(1, _HID), ln2_g.reshape(1, _HID), ln2_b.reshape(1, _HID),
      lin_w, lin_b.reshape(1, 1))
    return (output, dur3[:, :, 0])


# speedup vs baseline: 15.9996x; 1.0661x over previous
import jax
import jax.numpy as jnp
from jax import lax
from jax.experimental import pallas as pl
from jax.experimental.pallas import tpu as pltpu

_EMB = 256
_HID = 256
_L = 512
_B = 4
_T = 3600
_TB = 1800
_NT = _T // _TB
_W = 256


def _layer_norm(h, g, b):
    mu = jnp.mean(h, axis=1, keepdims=True)
    d = h - mu
    var = jnp.mean(d * d, axis=1, keepdims=True)
    return d * lax.rsqrt(var + 1e-5) * g + b


def _shift(h, z):
    prev = jnp.concatenate([z, h[:-1, :]], axis=0)
    nxt = jnp.concatenate([h[1:, :], z], axis=0)
    return prev, nxt


def _body(mel_ref, x_ref, tgt_ref, wk1_ref, b1_ref, g1_ref, be1_ref,
          wk2_ref, b2_ref, g2_ref, be2_ref, lw_ref, lb_ref,
          out_ref, dur_ref, st_s, en_s, xbf_s):
    jt = pl.program_id(1)

    @pl.when(jt == 0)
    def _():
        xb = x_ref[0]
        xbf_s[...] = xb.astype(jnp.bfloat16)
        z = jnp.zeros((1, _EMB), jnp.float32)
        xp, xn = _shift(xb, z)
        h = (jnp.dot(xp, wk1_ref[0], preferred_element_type=jnp.float32)
             + jnp.dot(xb, wk1_ref[1], preferred_element_type=jnp.float32)
             + jnp.dot(xn, wk1_ref[2], preferred_element_type=jnp.float32)
             + b1_ref[...])
        h = jax.nn.relu(_layer_norm(h, g1_ref[...], be1_ref[...]))
        hp, hn = _shift(h, z)
        h = (jnp.dot(hp, wk2_ref[0], preferred_element_type=jnp.float32)
             + jnp.dot(h, wk2_ref[1], preferred_element_type=jnp.float32)
             + jnp.dot(hn, wk2_ref[2], preferred_element_type=jnp.float32)
             + b2_ref[...])
        h = jax.nn.relu(_layer_norm(h, g2_ref[...], be2_ref[...]))
        dur = jax.nn.relu(jnp.dot(h, lw_ref[...],
                                  preferred_element_type=jnp.float32)
                          + lb_ref[...])
        dur_ref[0] = dur

        rr = lax.broadcasted_iota(jnp.int32, (_L, _L), 0)
        cc = lax.broadcasted_iota(jnp.int32, (_L, _L), 1)
        ut = (rr <= cc).astype(jnp.float32)
        tgt_row = tgt_ref[0].astype(jnp.float32)
        ends_row = jnp.dot(tgt_row, ut,
                           preferred_element_type=jnp.float32)
        ends_i = ends_row.astype(jnp.int32)
        en_s[...] = ends_i
        st_s[...] = ends_i - tgt_ref[0]

    mel = mel_ref[0, 0]
    starts_i = st_s[...]
    ends_i = en_s[...]
    t0 = jt * _TB
    t1 = t0 + _TB
    lo = jnp.sum((ends_i <= t0).astype(jnp.int32))
    hi = jnp.sum((starts_i < t1).astype(jnp.int32))
    shift = jnp.remainder(_L - lo, _L)
    t_col = lax.broadcasted_iota(jnp.int32, (_TB, 1), 0) + t0

    @pl.when(hi - lo <= _W)
    def _():
        st_w = pltpu.roll(starts_i, shift, axis=1)[:, :_W]
        en_w = jnp.minimum(pltpu.roll(ends_i, shift, axis=1)[:, :_W], mel)
        a_blk = ((st_w <= t_col) & (t_col < en_w)).astype(jnp.bfloat16)
        x_w = pltpu.roll(xbf_s[...], shift, axis=0)[:_W, :]
        out_ref[0] = jnp.dot(a_blk, x_w, preferred_element_type=jnp.float32)

    @pl.when(hi - lo > _W)
    def _():
        ends_eff = jnp.minimum(ends_i, mel)
        a_blk = ((starts_i <= t_col) & (t_col < ends_eff)).astype(jnp.bfloat16)
        out_ref[0] = jnp.dot(a_blk, xbf_s[...],
                             preferred_element_type=jnp.float32)


def kernel(x, target, mel_max_len, conv1_w, conv1_b, ln1_g, ln1_b,
           conv2_w, conv2_b, ln2_g, ln2_b, lin_w, lin_b):
    mel = jnp.asarray(mel_max_len, jnp.int32).reshape(1, 1)
    tgt3 = target.astype(jnp.int32).reshape(_B, 1, _L)
    wk1 = jnp.transpose(conv1_w, (2, 1, 0))
    wk2 = jnp.transpose(conv2_w, (2, 1, 0))
    full3 = lambda s: pl.BlockSpec(s, lambda b, jt: (0, 0, 0))
    full2 = lambda s: pl.BlockSpec(s, lambda b, jt: (0, 0))
    output, dur3 = pl.pallas_call(
        _body,
        grid=(_B, _NT),
        in_specs=[
            pl.BlockSpec(memory_space=pltpu.SMEM),
            pl.BlockSpec((1, _L, _EMB), lambda b, jt: (b, 0, 0)),
            pl.BlockSpec((1, 1, _L), lambda b, jt: (b, 0, 0)),
            full3((3, _EMB, _HID)),
            full2((1, _HID)), full2((1, _HID)), full2((1, _HID)),
            full3((3, _HID, _HID)),
            full2((1, _HID)), full2((1, _HID)), full2((1, _HID)),
            full2((_HID, 1)),
            full2((1, 1)),
        ],
        out_specs=[
            pl.BlockSpec((1, _TB, _EMB), lambda b, jt: (b, jt, 0)),
            pl.BlockSpec((1, _L, 1), lambda b, jt: (b, 0, 0)),
        ],
        out_shape=[
            jax.ShapeDtypeStruct((_B, _T, _EMB), jnp.float32),
            jax.ShapeDtypeStruct((_B, _L, 1), jnp.float32),
        ],
        scratch_shapes=[
            pltpu.VMEM((1, _L), jnp.int32),
            pltpu.VMEM((1, _L), jnp.int32),
            pltpu.VMEM((_L, _EMB), jnp.bfloat16),
        ],
    )(mel, x, tgt3, wk1,
      conv1_b.reshape(1, _HID), ln1_g.reshape(1, _HID), ln1_b.reshape(1, _HID),
      wk2,
      conv2_b.reshape(1, _HID), ln2_g.reshape(1, _HID), ln2_b.reshape(1, _HID),
      lin_w, lin_b.reshape(1, 1))
    return (output, dur3[:, :, 0])


# speedup vs baseline: 17.7849x; 1.1116x over previous
import jax
import jax.numpy as jnp
from jax import lax
from jax.experimental import pallas as pl
from jax.experimental.pallas import tpu as pltpu

_EMB = 256
_HID = 256
_L = 512
_B = 4
_T = 3600
_TB = 3600
_NT = _T // _TB
_W = 256


def _layer_norm(h, g, b):
    mu = jnp.mean(h, axis=1, keepdims=True)
    d = h - mu
    var = jnp.mean(d * d, axis=1, keepdims=True)
    return d * lax.rsqrt(var + 1e-5) * g + b


def _shift(h, z):
    prev = jnp.concatenate([z, h[:-1, :]], axis=0)
    nxt = jnp.concatenate([h[1:, :], z], axis=0)
    return prev, nxt


def _body(mel_ref, x_ref, tgt_ref, wk1_ref, b1_ref, g1_ref, be1_ref,
          wk2_ref, b2_ref, g2_ref, be2_ref, lw_ref, lb_ref,
          out_ref, dur_ref, st_s, en_s, xbf_s):
    jt = pl.program_id(1)

    @pl.when(jt == 0)
    def _():
        xb = x_ref[0]
        xbf_s[...] = xb.astype(jnp.bfloat16)
        z = jnp.zeros((1, _EMB), jnp.float32)
        xp, xn = _shift(xb, z)
        h = (jnp.dot(xp, wk1_ref[0], preferred_element_type=jnp.float32)
             + jnp.dot(xb, wk1_ref[1], preferred_element_type=jnp.float32)
             + jnp.dot(xn, wk1_ref[2], preferred_element_type=jnp.float32)
             + b1_ref[...])
        h = jax.nn.relu(_layer_norm(h, g1_ref[...], be1_ref[...]))
        hp, hn = _shift(h, z)
        h = (jnp.dot(hp, wk2_ref[0], preferred_element_type=jnp.float32)
             + jnp.dot(h, wk2_ref[1], preferred_element_type=jnp.float32)
             + jnp.dot(hn, wk2_ref[2], preferred_element_type=jnp.float32)
             + b2_ref[...])
        h = jax.nn.relu(_layer_norm(h, g2_ref[...], be2_ref[...]))
        dur = jax.nn.relu(jnp.dot(h, lw_ref[...],
                                  preferred_element_type=jnp.float32)
                          + lb_ref[...])
        dur_ref[0] = dur

        rr = lax.broadcasted_iota(jnp.int32, (_L, _L), 0)
        cc = lax.broadcasted_iota(jnp.int32, (_L, _L), 1)
        ut = (rr <= cc).astype(jnp.float32)
        tgt_row = tgt_ref[0].astype(jnp.float32)
        ends_row = jnp.dot(tgt_row, ut,
                           preferred_element_type=jnp.float32)
        ends_i = ends_row.astype(jnp.int32)
        en_s[...] = ends_i
        st_s[...] = ends_i - tgt_ref[0]

    mel = mel_ref[0, 0]
    starts_i = st_s[...]
    ends_i = en_s[...]
    t0 = jt * _TB
    t1 = t0 + _TB
    lo = jnp.sum((ends_i <= t0).astype(jnp.int32))
    hi = jnp.sum((starts_i < t1).astype(jnp.int32))
    shift = jnp.remainder(_L - lo, _L)
    t_col = lax.broadcasted_iota(jnp.int32, (_TB, 1), 0) + t0

    @pl.when(hi - lo <= _W)
    def _():
        st_w = pltpu.roll(starts_i, shift, axis=1)[:, :_W]
        en_w = jnp.minimum(pltpu.roll(ends_i, shift, axis=1)[:, :_W], mel)
        a_blk = ((st_w <= t_col) & (t_col < en_w)).astype(jnp.bfloat16)
        x_w = pltpu.roll(xbf_s[...], shift, axis=0)[:_W, :]
        out_ref[0] = jnp.dot(a_blk, x_w, preferred_element_type=jnp.float32)

    @pl.when(hi - lo > _W)
    def _():
        ends_eff = jnp.minimum(ends_i, mel)
        a_blk = ((starts_i <= t_col) & (t_col < ends_eff)).astype(jnp.bfloat16)
        out_ref[0] = jnp.dot(a_blk, xbf_s[...],
                             preferred_element_type=jnp.float32)


def kernel(x, target, mel_max_len, conv1_w, conv1_b, ln1_g, ln1_b,
           conv2_w, conv2_b, ln2_g, ln2_b, lin_w, lin_b):
    mel = jnp.asarray(mel_max_len, jnp.int32).reshape(1, 1)
    tgt3 = target.astype(jnp.int32).reshape(_B, 1, _L)
    wk1 = jnp.transpose(conv1_w, (2, 1, 0))
    wk2 = jnp.transpose(conv2_w, (2, 1, 0))
    full3 = lambda s: pl.BlockSpec(s, lambda b, jt: (0, 0, 0))
    full2 = lambda s: pl.BlockSpec(s, lambda b, jt: (0, 0))
    output, dur3 = pl.pallas_call(
        _body,
        grid=(_B, _NT),
        in_specs=[
            pl.BlockSpec(memory_space=pltpu.SMEM),
            pl.BlockSpec((1, _L, _EMB), lambda b, jt: (b, 0, 0)),
            pl.BlockSpec((1, 1, _L), lambda b, jt: (b, 0, 0)),
            full3((3, _EMB, _HID)),
            full2((1, _HID)), full2((1, _HID)), full2((1, _HID)),
            full3((3, _HID, _HID)),
            full2((1, _HID)), full2((1, _HID)), full2((1, _HID)),
            full2((_HID, 1)),
            full2((1, 1)),
        ],
        out_specs=[
            pl.BlockSpec((1, _TB, _EMB), lambda b, jt: (b, jt, 0)),
            pl.BlockSpec((1, _L, 1), lambda b, jt: (b, 0, 0)),
        ],
        out_shape=[
            jax.ShapeDtypeStruct((_B, _T, _EMB), jnp.float32),
            jax.ShapeDtypeStruct((_B, _L, 1), jnp.float32),
        ],
        scratch_shapes=[
            pltpu.VMEM((1, _L), jnp.int32),
            pltpu.VMEM((1, _L), jnp.int32),
            pltpu.VMEM((_L, _EMB), jnp.bfloat16),
        ],
    )(mel, x, tgt3, wk1,
      conv1_b.reshape(1, _HID), ln1_g.reshape(1, _HID), ln1_b.reshape(1, _HID),
      wk2,
      conv2_b.reshape(1, _HID), ln2_g.reshape(1, _HID), ln2_b.reshape(1, _HID),
      lin_w, lin_b.reshape(1, 1))
    return (output, dur3[:, :, 0])
